# Initial kernel scaffold; baseline (speedup 1.0000x reference)
import jax
import jax.numpy as jnp
from jax.experimental import pallas as pl


def kernel(x, emb_table):
    raise NotImplementedError("write your pallas kernel here")


# speedup vs baseline: 1.1182x; 1.1182x over previous
import functools

import jax
import jax.numpy as jnp
from jax import lax
from jax.experimental import pallas as pl
from jax.experimental.pallas import tpu as pltpu
from jax.experimental.pallas import tpu_sc as plsc

_NC = 2
_NS = 16
_NW = _NC * _NS
_CHUNK = 32


@functools.partial(jax.jit, static_argnums=(1, 2, 3))
def _broadcast_rows(table_flat, batch, max_len, d):
    rows_per_w = max_len // _NW
    n_chunks = rows_per_w // _CHUNK
    chunk_elems = _CHUNK * d

    def body(table_hbm, out_hbm, buf0, buf1, in_sem, out_sem):
        c = lax.axis_index("c")
        s = lax.axis_index("s")
        wid = s * _NC + c
        base = wid * rows_per_w
        bufs = (buf0, buf1)

        def in_copy(i):
            return pltpu.make_async_copy(
                table_hbm.at[pl.ds((base + i * _CHUNK + 1) * d, chunk_elems)],
                bufs[i % 2], in_sem)

        def out_copies(i):
            row0 = base + i * _CHUNK
            return [
                pltpu.make_async_copy(
                    bufs[i % 2],
                    out_hbm.at[pl.ds((b * max_len + row0) * d, chunk_elems)],
                    out_sem)
                for b in range(batch)
            ]

        pending_in = in_copy(0)
        pending_in.start()
        pending_out = []
        for i in range(n_chunks):
            pending_in.wait()
            outs = out_copies(i)
            for cp in outs:
                cp.start()
            if i + 1 < n_chunks:
                for cp in pending_out:
                    cp.wait()
                pending_out = outs
                pending_in = in_copy(i + 1)
                pending_in.start()
            else:
                for cp in pending_out:
                    cp.wait()
                for cp in outs:
                    cp.wait()

    return pl.kernel(
        body,
        out_type=jax.ShapeDtypeStruct((batch * max_len * d,), table_flat.dtype),
        mesh=plsc.VectorSubcoreMesh(
            core_axis_name="c", subcore_axis_name="s",
            num_cores=_NC, num_subcores=_NS),
        scratch_types=[
            pltpu.VMEM((chunk_elems,), table_flat.dtype),
            pltpu.VMEM((chunk_elems,), table_flat.dtype),
            pltpu.SemaphoreType.DMA,
            pltpu.SemaphoreType.DMA,
        ],
    )(table_flat)


def kernel(x, emb_table):
    batch, max_len = x.shape
    d = emb_table.shape[1]
    flat = _broadcast_rows(emb_table.reshape(-1), batch, max_len, d)
    return flat.reshape(batch, max_len, d)
